# baseline (speedup 1.0000x reference)
import functools

import jax
import jax.numpy as jnp
from jax.experimental import pallas as pl
from jax.experimental.pallas import tpu as pltpu

LOG_SIG_MAX = 2.0
LOG_SIG_MIN = -20.0

W = 32


def _pick_block(total, target):
    b = min(target, total)
    while total % b:
        b -= 1
    return b


def _mlp_agg(nwin, lo, hi, seg_ref, mm, table_ref, acc_ref, cnt_ref):
    ids = seg_ref[0]
    base8 = (lo // 8) * 8
    rel = ids - base8
    iota = jax.lax.broadcasted_iota(jnp.int32, (W, 1), 0)
    dn = (((0,), (0,)), ((), ()))

    def _window(w):
        oh = (rel == iota + (w * W)).astype(jnp.float32)
        start = base8 + w * W
        tab = table_ref[pl.ds(start, W)]
        g = jax.lax.dot_general(oh, tab, dn,
                                preferred_element_type=jnp.float32)
        act = jnp.maximum(mm + g, 0.0)
        acc_ref[pl.ds(start, W)] += jnp.dot(
            oh, act, preferred_element_type=jnp.float32)
        cnt_ref[pl.ds(start, W)] += jnp.sum(oh, axis=1, keepdims=True)

    _window(0)
    for w in range(1, nwin):
        @pl.when(hi >= base8 + w * W)
        def _(w=w):
            _window(w)


def _edge_body(nwin, nstream, lo_ref, hi_ref, *refs):
    seg_refs = refs[:nstream]
    eblk_refs = refs[nstream:2 * nstream]
    (u_ref, Wue_ref, be_ref, We_ref, acc_out_ref, cnt_out_ref) = \
        refs[2 * nstream:2 * nstream + 6]
    scratch = refs[2 * nstream + 6:]
    ue_ref = scratch[0]
    acc_refs = scratch[1:1 + nstream]
    cnt_refs = scratch[1 + nstream:1 + 2 * nstream]

    i = pl.program_id(0)
    nb = pl.num_programs(0)
    Bg = acc_out_ref.shape[0]

    @pl.when(i == 0)
    def _init():
        ue_ref[...] = jnp.zeros_like(ue_ref)
        ue_ref[:Bg] = (jnp.dot(u_ref[...], Wue_ref[...],
                               preferred_element_type=jnp.float32)
                       + be_ref[...])
        for s in range(nstream):
            acc_refs[s][...] = jnp.zeros_like(acc_refs[s])
            cnt_refs[s][...] = jnp.zeros_like(cnt_refs[s])

    for s in range(nstream):
        mm = jnp.dot(eblk_refs[s][...], We_ref[...],
                     preferred_element_type=jnp.float32)
        _mlp_agg(nwin, lo_ref[s * nb + i], hi_ref[s * nb + i],
                 seg_refs[s], mm, ue_ref, acc_refs[s], cnt_refs[s])

    @pl.when(i == nb - 1)
    def _emit():
        acc = acc_refs[0][:Bg]
        cnt = cnt_refs[0][:Bg]
        for s in range(1, nstream):
            acc = acc + acc_refs[s][:Bg]
            cnt = cnt + cnt_refs[s][:Bg]
        acc_out_ref[...] = acc
        cnt_out_ref[...] = cnt


def _node_body(nwin, lo_ref, hi_ref, seg_ref, xblk_ref, u_ref, Wun_ref, bn_ref,
               Wn_ref, acc_e_ref, cnt_e_ref,
               Wgn_m_ref, Wge_m_ref, bg_m_ref,
               Wgn_s_ref, Wge_s_ref, bg_s_ref,
               mean_ref, logstd_ref,
               un_ref, acc_ref, cnt_ref):
    i = pl.program_id(0)
    nb = pl.num_programs(0)
    Bg = acc_e_ref.shape[0]

    @pl.when(i == 0)
    def _init():
        un_ref[...] = jnp.zeros_like(un_ref)
        un_ref[:Bg] = (jnp.dot(u_ref[...], Wun_ref[...],
                               preferred_element_type=jnp.float32)
                       + bn_ref[...])
        acc_ref[...] = jnp.zeros_like(acc_ref)
        cnt_ref[...] = jnp.zeros_like(cnt_ref)

    mm = jnp.dot(xblk_ref[...], Wn_ref[...],
                 preferred_element_type=jnp.float32)
    _mlp_agg(nwin, lo_ref[i], hi_ref[i], seg_ref, mm, un_ref, acc_ref, cnt_ref)

    @pl.when(i == nb - 1)
    def _finish():
        n_agg = acc_ref[:Bg] / jnp.maximum(cnt_ref[:Bg], 1.0)
        e_agg = acc_e_ref[...] / jnp.maximum(cnt_e_ref[...], 1.0)
        mean_ref[...] = (
            jnp.dot(n_agg, Wgn_m_ref[...], preferred_element_type=jnp.float32)
            + jnp.dot(e_agg, Wge_m_ref[...], preferred_element_type=jnp.float32)
            + bg_m_ref[...])
        ls = (jnp.dot(n_agg, Wgn_s_ref[...], preferred_element_type=jnp.float32)
              + jnp.dot(e_agg, Wge_s_ref[...], preferred_element_type=jnp.float32)
              + bg_s_ref[...])
        logstd_ref[...] = jnp.clip(ls, LOG_SIG_MIN, LOG_SIG_MAX)


def kernel(x, edge_attr, u, node2graph, edge2graph, We, Wue, be, Wn, Wun, bn,
           Wgn_m, Wge_m, bg_m, Wgn_s, Wge_s, bg_s):
    N, DN = x.shape
    E, DE = edge_attr.shape
    B, DU = u.shape
    H = We.shape[1]
    A = Wgn_m.shape[1]
    f32 = jnp.float32

    BE = _pick_block(E, 10000)
    KE = E // BE
    BN = _pick_block(N, 5000)
    KN = N // BN
    nwin = -(-B // W)
    Bpad = -(-B // 8) * 8 + nwin * W
    nstream = 2 if KE % 2 == 0 else 1
    KS = KE // nstream

    e2g = edge2graph.reshape(KE, 1, BE)
    n2g = node2graph.reshape(KN, 1, BN)
    e_lo = edge2graph[0::BE]
    e_hi = edge2graph[BE - 1::BE]
    n_lo = node2graph[0::BN]
    n_hi = node2graph[BN - 1::BN]
    be2 = be.reshape(1, H)
    bn2 = bn.reshape(1, H)
    bgm2 = bg_m.reshape(1, A)
    bgs2 = bg_s.reshape(1, A)

    full = lambda shape: pl.BlockSpec(shape, lambda i: (0,) * len(shape))
    smem = pl.BlockSpec(memory_space=pltpu.SMEM)

    seg_specs = [pl.BlockSpec((1, 1, BE), functools.partial(
        lambda s, i: (i + s * KS, 0, 0), s)) for s in range(nstream)]
    eblk_specs = [pl.BlockSpec((BE, DE), functools.partial(
        lambda s, i: (i + s * KS, 0), s)) for s in range(nstream)]

    acc_e, cnt_e = pl.pallas_call(
        functools.partial(_edge_body, nwin, nstream),
        grid=(KS,),
        in_specs=[smem, smem] + seg_specs + eblk_specs + [
            full((B, DU)),
            full((DU, H)),
            full((1, H)),
            full((DE, H)),
        ],
        out_specs=[full((B, H)), full((B, 1))],
        out_shape=[jax.ShapeDtypeStruct((B, H), f32),
                   jax.ShapeDtypeStruct((B, 1), f32)],
        scratch_shapes=[pltpu.VMEM((Bpad, H), f32)]
        + [pltpu.VMEM((Bpad, H), f32) for _ in range(nstream)]
        + [pltpu.VMEM((Bpad, 1), f32) for _ in range(nstream)],
    )(e_lo, e_hi, *([e2g] * nstream), *([edge_attr] * nstream),
      u, Wue, be2, We)

    mean, log_std = pl.pallas_call(
        functools.partial(_node_body, nwin),
        grid=(KN,),
        in_specs=[
            smem,
            smem,
            pl.BlockSpec((1, 1, BN), lambda i: (i, 0, 0)),
            pl.BlockSpec((BN, DN), lambda i: (i, 0)),
            full((B, DU)),
            full((DU, H)),
            full((1, H)),
            full((DN, H)),
            full((B, H)),
            full((B, 1)),
            full((H, A)),
            full((H, A)),
            full((1, A)),
            full((H, A)),
            full((H, A)),
            full((1, A)),
        ],
        out_specs=[full((B, A)), full((B, A))],
        out_shape=[jax.ShapeDtypeStruct((B, A), f32),
                   jax.ShapeDtypeStruct((B, A), f32)],
        scratch_shapes=[pltpu.VMEM((Bpad, H), f32),
                        pltpu.VMEM((Bpad, H), f32),
                        pltpu.VMEM((Bpad, 1), f32)],
    )(n_lo, n_hi, n2g, x, u, Wun, bn2, Wn, acc_e, cnt_e,
      Wgn_m, Wge_m, bgm2, Wgn_s, Wge_s, bgs2)

    return (mean, log_std)


# speedup vs baseline: 1.1153x; 1.1153x over previous
import functools

import jax
import jax.numpy as jnp
from jax.experimental import pallas as pl
from jax.experimental.pallas import tpu as pltpu

LOG_SIG_MAX = 2.0
LOG_SIG_MIN = -20.0

W = 64


def _pick_block(total, target):
    b = min(target, total)
    while total % b:
        b -= 1
    return b


def _mlp_agg(nwin, lo, hi, seg_ref, mm, table_ref, acc_ref, cnt_ref):
    ids = seg_ref[0]
    base8 = (lo // 8) * 8
    rel = ids - base8
    iota = jax.lax.broadcasted_iota(jnp.int32, (W, 1), 0)
    dn = (((0,), (0,)), ((), ()))

    def _window(w):
        oh = (rel == iota + (w * W)).astype(jnp.float32)
        start = base8 + w * W
        tab = table_ref[pl.ds(start, W)]
        g = jax.lax.dot_general(oh, tab, dn,
                                preferred_element_type=jnp.float32)
        act = jnp.maximum(mm + g, 0.0)
        acc_ref[pl.ds(start, W)] += jnp.dot(
            oh, act, preferred_element_type=jnp.float32)
        cnt_ref[pl.ds(start, W)] += jnp.sum(oh, axis=1, keepdims=True)

    _window(0)
    for w in range(1, nwin):
        @pl.when(hi >= base8 + w * W)
        def _(w=w):
            _window(w)


def _edge_body(nwin, nstream, lo_ref, hi_ref, *refs):
    seg_refs = refs[:nstream]
    eblk_refs = refs[nstream:2 * nstream]
    (u_ref, Wue_ref, be_ref, We_ref, acc_out_ref, cnt_out_ref) = \
        refs[2 * nstream:2 * nstream + 6]
    scratch = refs[2 * nstream + 6:]
    ue_ref = scratch[0]
    acc_refs = scratch[1:1 + nstream]
    cnt_refs = scratch[1 + nstream:1 + 2 * nstream]

    i = pl.program_id(0)
    nb = pl.num_programs(0)
    Bg = acc_out_ref.shape[0]

    @pl.when(i == 0)
    def _init():
        ue_ref[...] = jnp.zeros_like(ue_ref)
        ue_ref[:Bg] = (jnp.dot(u_ref[...], Wue_ref[...],
                               preferred_element_type=jnp.float32)
                       + be_ref[...])
        for s in range(nstream):
            acc_refs[s][...] = jnp.zeros_like(acc_refs[s])
            cnt_refs[s][...] = jnp.zeros_like(cnt_refs[s])

    for s in range(nstream):
        mm = jnp.dot(eblk_refs[s][...], We_ref[...],
                     preferred_element_type=jnp.float32)
        _mlp_agg(nwin, lo_ref[s * nb + i], hi_ref[s * nb + i],
                 seg_refs[s], mm, ue_ref, acc_refs[s], cnt_refs[s])

    @pl.when(i == nb - 1)
    def _emit():
        acc = acc_refs[0][:Bg]
        cnt = cnt_refs[0][:Bg]
        for s in range(1, nstream):
            acc = acc + acc_refs[s][:Bg]
            cnt = cnt + cnt_refs[s][:Bg]
        acc_out_ref[...] = acc
        cnt_out_ref[...] = cnt


def _node_body(nwin, lo_ref, hi_ref, seg_ref, xblk_ref, u_ref, Wun_ref, bn_ref,
               Wn_ref, acc_e_ref, cnt_e_ref,
               Wgn_m_ref, Wge_m_ref, bg_m_ref,
               Wgn_s_ref, Wge_s_ref, bg_s_ref,
               mean_ref, logstd_ref,
               un_ref, acc_ref, cnt_ref):
    i = pl.program_id(0)
    nb = pl.num_programs(0)
    Bg = acc_e_ref.shape[0]

    @pl.when(i == 0)
    def _init():
        un_ref[...] = jnp.zeros_like(un_ref)
        un_ref[:Bg] = (jnp.dot(u_ref[...], Wun_ref[...],
                               preferred_element_type=jnp.float32)
                       + bn_ref[...])
        acc_ref[...] = jnp.zeros_like(acc_ref)
        cnt_ref[...] = jnp.zeros_like(cnt_ref)

    mm = jnp.dot(xblk_ref[...], Wn_ref[...],
                 preferred_element_type=jnp.float32)
    _mlp_agg(nwin, lo_ref[i], hi_ref[i], seg_ref, mm, un_ref, acc_ref, cnt_ref)

    @pl.when(i == nb - 1)
    def _finish():
        n_agg = acc_ref[:Bg] / jnp.maximum(cnt_ref[:Bg], 1.0)
        e_agg = acc_e_ref[...] / jnp.maximum(cnt_e_ref[...], 1.0)
        mean_ref[...] = (
            jnp.dot(n_agg, Wgn_m_ref[...], preferred_element_type=jnp.float32)
            + jnp.dot(e_agg, Wge_m_ref[...], preferred_element_type=jnp.float32)
            + bg_m_ref[...])
        ls = (jnp.dot(n_agg, Wgn_s_ref[...], preferred_element_type=jnp.float32)
              + jnp.dot(e_agg, Wge_s_ref[...], preferred_element_type=jnp.float32)
              + bg_s_ref[...])
        logstd_ref[...] = jnp.clip(ls, LOG_SIG_MIN, LOG_SIG_MAX)


def kernel(x, edge_attr, u, node2graph, edge2graph, We, Wue, be, Wn, Wun, bn,
           Wgn_m, Wge_m, bg_m, Wgn_s, Wge_s, bg_s):
    N, DN = x.shape
    E, DE = edge_attr.shape
    B, DU = u.shape
    H = We.shape[1]
    A = Wgn_m.shape[1]
    f32 = jnp.float32

    BE = _pick_block(E, 16000)
    KE = E // BE
    BN = _pick_block(N, 5000)
    KN = N // BN
    nwin = -(-B // W)
    Bpad = -(-B // 8) * 8 + nwin * W
    nstream = 2 if KE % 2 == 0 else 1
    KS = KE // nstream

    e2g = edge2graph.reshape(KE, 1, BE)
    n2g = node2graph.reshape(KN, 1, BN)
    e_lo = edge2graph[0::BE]
    e_hi = edge2graph[BE - 1::BE]
    n_lo = node2graph[0::BN]
    n_hi = node2graph[BN - 1::BN]
    be2 = be.reshape(1, H)
    bn2 = bn.reshape(1, H)
    bgm2 = bg_m.reshape(1, A)
    bgs2 = bg_s.reshape(1, A)

    full = lambda shape: pl.BlockSpec(shape, lambda i: (0,) * len(shape))
    smem = pl.BlockSpec(memory_space=pltpu.SMEM)

    seg_specs = [pl.BlockSpec((1, 1, BE), functools.partial(
        lambda s, i: (i + s * KS, 0, 0), s)) for s in range(nstream)]
    eblk_specs = [pl.BlockSpec((BE, DE), functools.partial(
        lambda s, i: (i + s * KS, 0), s)) for s in range(nstream)]

    acc_e, cnt_e = pl.pallas_call(
        functools.partial(_edge_body, nwin, nstream),
        grid=(KS,),
        in_specs=[smem, smem] + seg_specs + eblk_specs + [
            full((B, DU)),
            full((DU, H)),
            full((1, H)),
            full((DE, H)),
        ],
        out_specs=[full((B, H)), full((B, 1))],
        out_shape=[jax.ShapeDtypeStruct((B, H), f32),
                   jax.ShapeDtypeStruct((B, 1), f32)],
        scratch_shapes=[pltpu.VMEM((Bpad, H), f32)]
        + [pltpu.VMEM((Bpad, H), f32) for _ in range(nstream)]
        + [pltpu.VMEM((Bpad, 1), f32) for _ in range(nstream)],
    )(e_lo, e_hi, *([e2g] * nstream), *([edge_attr] * nstream),
      u, Wue, be2, We)

    mean, log_std = pl.pallas_call(
        functools.partial(_node_body, nwin),
        grid=(KN,),
        in_specs=[
            smem,
            smem,
            pl.BlockSpec((1, 1, BN), lambda i: (i, 0, 0)),
            pl.BlockSpec((BN, DN), lambda i: (i, 0)),
            full((B, DU)),
            full((DU, H)),
            full((1, H)),
            full((DN, H)),
            full((B, H)),
            full((B, 1)),
            full((H, A)),
            full((H, A)),
            full((1, A)),
            full((H, A)),
            full((H, A)),
            full((1, A)),
        ],
        out_specs=[full((B, A)), full((B, A))],
        out_shape=[jax.ShapeDtypeStruct((B, A), f32),
                   jax.ShapeDtypeStruct((B, A), f32)],
        scratch_shapes=[pltpu.VMEM((Bpad, H), f32),
                        pltpu.VMEM((Bpad, H), f32),
                        pltpu.VMEM((Bpad, 1), f32)],
    )(n_lo, n_hi, n2g, x, u, Wun, bn2, Wn, acc_e, cnt_e,
      Wgn_m, Wge_m, bgm2, Wgn_s, Wge_s, bgs2)

    return (mean, log_std)
